# baseline (speedup 1.0000x reference)
import jax
import jax.numpy as jnp
from jax import lax
from jax.experimental import pallas as pl
from jax.experimental.pallas import tpu as pltpu
from jax.experimental.pallas import tpu_sc as plsc

NUM_TRUST = 6
DIM = 64
LANES = 16
NUM_CORES = 2
NUM_SUBCORES = 16
NW = NUM_CORES * NUM_SUBCORES

NB = 2
SUBS = ((0, 104), (104, 96))
NBUF = 3


def _embed_body(types_hbm, trusts_hbm, type_table_hbm, trust_table_hbm,
                out_hbm,
                i0, i1, i2, t0, t1, t2, r0, r1, r2, trust_v,
                g0, g1, g2, o0, o1, o2):
    ibufs = (i0, i1, i2)
    tbufs = (t0, t1, t2)
    rows = (r0, r1, r2)
    gsems = (g0, g1, g2)
    osems = (o0, o1, o2)

    b, l = types_hbm.shape
    per_w = b // NW
    nch = per_w // NB
    wid = lax.axis_index("s") * NUM_CORES + lax.axis_index("c")
    wbase = wid * per_w

    pltpu.sync_copy(trust_table_hbm, trust_v)

    def gather_copies(r):
        cs = []
        for i in range(NB):
            for off, sz in SUBS:
                cs.append(pltpu.make_async_copy(
                    type_table_hbm.at[ibufs[r].at[i, pl.ds(off, sz)]],
                    rows[r].at[i, pl.ds(off, sz)],
                    gsems[r]))
        return cs

    def out_copy(k, r):
        base = pl.multiple_of(wbase + k * NB, NB)
        return pltpu.make_async_copy(rows[r], out_hbm.at[pl.ds(base, NB)],
                                     osems[r])

    def start(k, r, wait_out):
        base = pl.multiple_of(wbase + k * NB, NB)
        pltpu.sync_copy(types_hbm.at[pl.ds(base, NB)], ibufs[r])
        pltpu.sync_copy(trusts_hbm.at[pl.ds(base, NB)], tbufs[r])
        if wait_out == "always":
            out_copy(k - NBUF, r).wait()
        elif wait_out == "guarded":
            @pl.when(k >= NBUF)
            def _():
                out_copy(k - NBUF, r).wait()
        for c in gather_copies(r):
            c.start()

    def finish(k, r):
        for c in gather_copies(r):
            c.wait()

        def row_body(i, _):
            def group_body(g, _):
                t_vec = tbufs[r][i, pl.ds(g * LANES, LANES)]
                for lane in range(LANES):
                    t = t_vec[lane]
                    for db in range(DIM // LANES):
                        plsc.addupdate(
                            rows[r].at[i, g * LANES + lane,
                                       pl.ds(db * LANES, LANES)],
                            trust_v[t, pl.ds(db * LANES, LANES)])
                return 0

            lax.fori_loop(0, l // LANES, group_body, 0, unroll=False)

            rem = l % LANES
            if rem:
                t_vec = tbufs[r][i, pl.ds(l - LANES, LANES)]
                for lane in range(LANES - rem, LANES):
                    t = t_vec[lane]
                    pos = l - LANES + lane
                    for db in range(DIM // LANES):
                        plsc.addupdate(
                            rows[r].at[i, pos, pl.ds(db * LANES, LANES)],
                            trust_v[t, pl.ds(db * LANES, LANES)])
            return 0

        lax.fori_loop(0, NB, row_body, 0, unroll=False)
        out_copy(k, r).start()

    start(0, 0, wait_out="none")
    start(1, 1, wait_out="none")

    nsup = (nch - 2) // NBUF

    def super_body(s, _):
        for j in range(NBUF):
            k = s * NBUF + j
            ks = k + 2
            rs = (j + 2) % NBUF
            start(ks, rs, wait_out="guarded" if j == 0 else "always")
            finish(k, j)
        return 0

    lax.fori_loop(0, nsup, super_body, 0, unroll=False)

    for k in range(nsup * NBUF, nch):
        ks = k + 2
        if ks < nch:
            start(ks, ks % NBUF, wait_out="always" if ks >= NBUF else "none")
        finish(k, k % NBUF)

    for k in range(max(0, nch - NBUF), nch):
        out_copy(k, k % NBUF).wait()


@jax.jit
def kernel(node_types, node_trust, type_table, trust_table):
    b, l = node_types.shape
    assert b % (NW * NB) == 0 and sum(s for _, s in SUBS) == l, (b, l)

    mesh = plsc.VectorSubcoreMesh(
        core_axis_name="c", subcore_axis_name="s",
        num_cores=NUM_CORES, num_subcores=NUM_SUBCORES)

    run = pl.kernel(
        _embed_body,
        out_type=jax.ShapeDtypeStruct((b, l, DIM), jnp.float32),
        mesh=mesh,
        compiler_params=pltpu.CompilerParams(
            needs_layout_passes=False, use_tc_tiling_on_sc=False),
        scratch_types=(
            [pltpu.VMEM((NB, l), jnp.int32) for _ in range(NBUF)]
            + [pltpu.VMEM((NB, l), jnp.int32) for _ in range(NBUF)]
            + [pltpu.VMEM((NB, l, DIM), jnp.float32) for _ in range(NBUF)]
            + [pltpu.VMEM((NUM_TRUST, DIM), jnp.float32)]
            + [pltpu.SemaphoreType.DMA for _ in range(2 * NBUF)]
        ),
    )
    return run(node_types.astype(jnp.int32), node_trust.astype(jnp.int32),
               type_table, trust_table)


# speedup vs baseline: 1.3898x; 1.3898x over previous
import jax
import jax.numpy as jnp
from jax import lax
from jax.experimental import pallas as pl
from jax.experimental.pallas import tpu as pltpu
from jax.experimental.pallas import tpu_sc as plsc

NUM_TRUST = 6
DIM = 64
LANES = 16
NUM_CORES = 2
NUM_SUBCORES = 16
NW = NUM_CORES * NUM_SUBCORES

NB = 2
SUBS = ((0, 104), (104, 96))
NBUF = 3


def _embed_body(types_hbm, trusts_hbm, type_table_hbm, trust_table_hbm,
                out_hbm,
                i0, i1, i2, t0, t1, t2, r0, r1, r2, trust_v,
                g0, g1, g2, o0, o1, o2):
    ibufs = (i0, i1, i2)
    tbufs = (t0, t1, t2)
    rows = (r0, r1, r2)
    gsems = (g0, g1, g2)
    osems = (o0, o1, o2)

    b, l = types_hbm.shape
    per_w = b // NW
    nch = per_w // NB
    wid = lax.axis_index("s") * NUM_CORES + lax.axis_index("c")
    wbase = wid * per_w

    pltpu.sync_copy(trust_table_hbm, trust_v)

    def gather_copies(r):
        cs = []
        for i in range(NB):
            for off, sz in SUBS:
                cs.append(pltpu.make_async_copy(
                    type_table_hbm.at[ibufs[r].at[i, pl.ds(off, sz)]],
                    rows[r].at[i, pl.ds(off, sz)],
                    gsems[r]))
        return cs

    def out_copy(k, r):
        base = pl.multiple_of(wbase + k * NB, NB)
        return pltpu.make_async_copy(
            rows[r],
            out_hbm.at[pl.ds(base, NB), :, pl.ds(0, DIM)],
            osems[r])

    def start(k, r, wait_out):
        base = pl.multiple_of(wbase + k * NB, NB)
        pltpu.sync_copy(types_hbm.at[pl.ds(base, NB)], ibufs[r])
        pltpu.sync_copy(trusts_hbm.at[pl.ds(base, NB)], tbufs[r])
        if wait_out == "always":
            out_copy(k - NBUF, r).wait()
        elif wait_out == "guarded":
            @pl.when(k >= NBUF)
            def _():
                out_copy(k - NBUF, r).wait()
        for c in gather_copies(r):
            c.start()

    def finish(k, r):
        for c in gather_copies(r):
            c.wait()

        def row_body(i, _):
            def group_body(g, _):
                t_vec = tbufs[r][i, pl.ds(g * LANES, LANES)]
                for lane in range(LANES):
                    t = t_vec[lane]
                    for db in range(DIM // LANES):
                        plsc.addupdate(
                            rows[r].at[i, g * LANES + lane,
                                       pl.ds(db * LANES, LANES)],
                            trust_v[t, pl.ds(db * LANES, LANES)])
                return 0

            lax.fori_loop(0, l // LANES, group_body, 0, unroll=False)

            rem = l % LANES
            if rem:
                t_vec = tbufs[r][i, pl.ds(l - LANES, LANES)]
                for lane in range(LANES - rem, LANES):
                    t = t_vec[lane]
                    pos = l - LANES + lane
                    for db in range(DIM // LANES):
                        plsc.addupdate(
                            rows[r].at[i, pos, pl.ds(db * LANES, LANES)],
                            trust_v[t, pl.ds(db * LANES, LANES)])
            return 0

        lax.fori_loop(0, NB, row_body, 0, unroll=False)
        out_copy(k, r).start()

    start(0, 0, wait_out="none")
    start(1, 1, wait_out="none")

    nsup = (nch - 2) // NBUF

    def super_body(s, _):
        for j in range(NBUF):
            k = s * NBUF + j
            ks = k + 2
            rs = (j + 2) % NBUF
            start(ks, rs, wait_out="guarded" if j == 0 else "always")
            finish(k, j)
        return 0

    lax.fori_loop(0, nsup, super_body, 0, unroll=False)

    for k in range(nsup * NBUF, nch):
        ks = k + 2
        if ks < nch:
            start(ks, ks % NBUF, wait_out="always" if ks >= NBUF else "none")
        finish(k, k % NBUF)

    for k in range(max(0, nch - NBUF), nch):
        out_copy(k, k % NBUF).wait()


@jax.jit
def kernel(node_types, node_trust, type_table, trust_table):
    b, l = node_types.shape
    assert b % (NW * NB) == 0 and sum(s for _, s in SUBS) == l, (b, l)

    mesh = plsc.VectorSubcoreMesh(
        core_axis_name="c", subcore_axis_name="s",
        num_cores=NUM_CORES, num_subcores=NUM_SUBCORES)

    run = pl.kernel(
        _embed_body,
        out_type=jax.ShapeDtypeStruct((b, l, 2 * DIM), jnp.float32),
        mesh=mesh,
        compiler_params=pltpu.CompilerParams(
            needs_layout_passes=False, use_tc_tiling_on_sc=False),
        scratch_types=(
            [pltpu.VMEM((NB, l), jnp.int32) for _ in range(NBUF)]
            + [pltpu.VMEM((NB, l), jnp.int32) for _ in range(NBUF)]
            + [pltpu.VMEM((NB, l, DIM), jnp.float32) for _ in range(NBUF)]
            + [pltpu.VMEM((NUM_TRUST, DIM), jnp.float32)]
            + [pltpu.SemaphoreType.DMA for _ in range(2 * NBUF)]
        ),
    )
    out = run(node_types.astype(jnp.int32), node_trust.astype(jnp.int32),
              type_table, trust_table)
    return out[:, :, :DIM]
